# baseline (speedup 1.0000x reference)
import math

import jax
import jax.numpy as jnp
from jax.experimental import pallas as pl
from jax.experimental.pallas import tpu as pltpu

_B, _S, _H, _E = 4, 2048, 1024, 16
_CHUNK = 256
_NSTEPS = _S // _CHUNK
_INV_SQRT2 = 1.0 / math.sqrt(2.0)


def _gate_kernel(x_ref, w1_ref, b1_ref, w2_ref, b2_ref, tw_ref, ti_ref, acc_ref):
    step = pl.program_id(0)

    @pl.when(step == 0)
    def _init():
        acc_ref[...] = jnp.zeros_like(acc_ref)

    acc_ref[0:_B, :] += jnp.sum(x_ref[...], axis=1)

    @pl.when(step == _NSTEPS - 1)
    def _tail():
        seq = acc_ref[...] * (1.0 / _S)
        h = jnp.dot(seq, w1_ref[...], preferred_element_type=jnp.float32)
        h = h + b1_ref[...]
        h = 0.5 * h * (1.0 + jax.lax.erf(h * _INV_SQRT2))
        logits = jnp.dot(h, w2_ref[...], preferred_element_type=jnp.float32)
        logits = logits + b2_ref[...]
        m = jnp.max(logits, axis=1, keepdims=True)
        ex = jnp.exp(logits - m)
        probs = ex / jnp.sum(ex, axis=1, keepdims=True)
        lane = jax.lax.broadcasted_iota(jnp.int32, probs.shape, 1)
        p1 = jnp.max(probs, axis=1, keepdims=True)
        i1 = jnp.min(jnp.where(probs == p1, lane, _E), axis=1, keepdims=True)
        masked = jnp.where(lane == i1, -1.0, probs)
        p2 = jnp.max(masked, axis=1, keepdims=True)
        i2 = jnp.min(jnp.where(masked == p2, lane, _E), axis=1, keepdims=True)
        e2 = jnp.exp(p2 - p1)
        denom = 1.0 + e2
        tw = jnp.concatenate([1.0 / denom, e2 / denom], axis=1)
        ti = jnp.concatenate([i1, i2], axis=1)
        tw_ref[...] = tw[0:_B, :]
        ti_ref[...] = ti[0:_B, :]


def kernel(hidden_states, W1, b1, W2, b2):
    tw, ti = pl.pallas_call(
        _gate_kernel,
        grid=(_NSTEPS,),
        in_specs=[
            pl.BlockSpec((_B, _CHUNK, _H), lambda i: (0, i, 0)),
            pl.BlockSpec((_H, _H), lambda i: (0, 0)),
            pl.BlockSpec((_H,), lambda i: (0,)),
            pl.BlockSpec((_H, _E), lambda i: (0, 0)),
            pl.BlockSpec((_E,), lambda i: (0,)),
        ],
        out_specs=[
            pl.BlockSpec((_B, 2), lambda i: (0, 0)),
            pl.BlockSpec((_B, 2), lambda i: (0, 0)),
        ],
        out_shape=[
            jax.ShapeDtypeStruct((_B, 2), jnp.float32),
            jax.ShapeDtypeStruct((_B, 2), jnp.int32),
        ],
        scratch_shapes=[pltpu.VMEM((8, _H), jnp.float32)],
    )(hidden_states, W1, b1, W2, b2)
    return tw, ti
